# baseline (speedup 1.0000x reference)
import functools

import jax
import jax.numpy as jnp
from jax import lax
from jax.experimental import pallas as pl
from jax.experimental.pallas import tpu as pltpu
from jax.experimental.pallas import tpu_sc as plsc

R, C, D = 16384, 200, 4
NC, NS, L = 2, 16, 16
NW = NC * NS
IB_ALL = R // 128
IB_W = IB_ALL // NW
CB = C // 8
MGRP = IB_W * 8
UNROLL = 2


def _compute_chunk(ids_ref, out_ref, tab_ref):
    @plsc.parallel_loop(0, 8 * MGRP, 1, unroll=UNROLL)
    def _(m):
        cl = lax.shift_right_logical(m, 5)
        ib = lax.bitwise_and(lax.shift_right_logical(m, 3), IB_W - 1)
        s = lax.bitwise_and(m, 7)
        idv = ids_ref[0, ib, cl, pl.ds(s * L, L)]
        a0 = lax.shift_left(idv, 2)
        for d in range(D):
            val = plsc.load_gather(tab_ref, [a0 + d]) + jnp.float32(0.02)
            out_ref[cl, ib, d, pl.ds(s * L, L)] = val


def _body(ids_hbm, tab_hbm, out_hbm,
          tab_v, ids_v0, ids_v1, out_v0, out_v1, out_v2,
          in_s0, in_s1, out_s0, out_s1, out_s2):
    wid = lax.axis_index("s") * NC + lax.axis_index("c")
    ib0 = wid * IB_W

    pltpu.sync_copy(tab_hbm, tab_v)

    ids_bufs = (ids_v0, ids_v1)
    out_bufs = (out_v0, out_v1, out_v2)
    in_sems = (in_s0, in_s1)
    out_sems = (out_s0, out_s1, out_s2)

    in_cp = {}
    out_cp = {}
    in_cp[0] = pltpu.async_copy(
        ids_hbm.at[pl.ds(0, 1), pl.ds(ib0, IB_W)], ids_bufs[0], in_sems[0])
    for g in range(CB):
        b = g & 1
        ob = g % 3
        in_cp[g].wait()
        if g + 1 < CB:
            in_cp[g + 1] = pltpu.async_copy(
                ids_hbm.at[pl.ds(g + 1, 1), pl.ds(ib0, IB_W)],
                ids_bufs[1 - b], in_sems[1 - b])
        if g >= 3:
            out_cp[g - 3].wait()
        _compute_chunk(ids_bufs[b], out_bufs[ob], tab_v)
        out_cp[g] = pltpu.async_copy(
            out_bufs[ob],
            out_hbm.at[pl.ds(g * 8, 8), pl.ds(ib0, IB_W)],
            out_sems[ob])
    out_cp[CB - 3].wait()
    out_cp[CB - 2].wait()
    out_cp[CB - 1].wait()


_sc_lookup = functools.partial(
    pl.kernel,
    out_type=jax.ShapeDtypeStruct((C, IB_ALL, D, 128), jnp.float32),
    mesh=plsc.VectorSubcoreMesh(
        core_axis_name="c", subcore_axis_name="s",
        num_cores=NC, num_subcores=NS),
    scratch_types=[
        pltpu.VMEM((8 * D,), jnp.float32),
        pltpu.VMEM((1, IB_W, 8, 128), jnp.int32),
        pltpu.VMEM((1, IB_W, 8, 128), jnp.int32),
        pltpu.VMEM((8, IB_W, D, 128), jnp.float32),
        pltpu.VMEM((8, IB_W, D, 128), jnp.float32),
        pltpu.VMEM((8, IB_W, D, 128), jnp.float32),
        pltpu.SemaphoreType.DMA,
        pltpu.SemaphoreType.DMA,
        pltpu.SemaphoreType.DMA,
        pltpu.SemaphoreType.DMA,
        pltpu.SemaphoreType.DMA,
    ],
    compiler_params=pltpu.CompilerParams(
        use_tc_tiling_on_sc=False, needs_layout_passes=False),
)(_body)


def kernel(input_ids, embed_tokens):
    ids4 = (input_ids.astype(jnp.int32).T
            .reshape(CB, 8, IB_ALL, 128).transpose(0, 2, 1, 3))
    tab_flat = embed_tokens.astype(jnp.float32).reshape(8 * D)
    z = _sc_lookup(ids4, tab_flat)
    return jnp.transpose(z, (1, 3, 0, 2)).reshape(R, C, D)


# speedup vs baseline: 1.0110x; 1.0110x over previous
import functools

import jax
import jax.numpy as jnp
from jax import lax
from jax.experimental import pallas as pl
from jax.experimental.pallas import tpu as pltpu
from jax.experimental.pallas import tpu_sc as plsc

R, C, D = 16384, 200, 4
NC, NS, L = 2, 16, 16
NW = NC * NS
IB_ALL = R // 128
IB_W = IB_ALL // NW
CB = C // 8
MGRP = IB_W * 8
UNROLL = 2


def _compute_chunk(ids_ref, out_ref, tab_ref):
    @plsc.parallel_loop(0, 8 * MGRP, 1, unroll=UNROLL)
    def _(m):
        cl = lax.shift_right_logical(m, 5)
        ib = lax.bitwise_and(lax.shift_right_logical(m, 3), IB_W - 1)
        s = lax.bitwise_and(m, 7)
        idv = ids_ref[0, ib, cl, pl.ds(s * L, L)]
        a0 = lax.shift_left(idv, 2)
        for d in range(D):
            out_ref[cl, ib, d, pl.ds(s * L, L)] = plsc.load_gather(
                tab_ref, [a0 + d])


def _body(ids_hbm, tab_hbm, out_hbm,
          tab_v, ids_v0, ids_v1, out_v0, out_v1, out_v2,
          in_s0, in_s1, out_s0, out_s1, out_s2):
    wid = lax.axis_index("s") * NC + lax.axis_index("c")
    ib0 = wid * IB_W

    pltpu.sync_copy(tab_hbm, tab_v)

    ids_bufs = (ids_v0, ids_v1)
    out_bufs = (out_v0, out_v1, out_v2)
    in_sems = (in_s0, in_s1)
    out_sems = (out_s0, out_s1, out_s2)

    in_cp = {}
    out_cp = {}
    in_cp[0] = pltpu.async_copy(
        ids_hbm.at[pl.ds(0, 1), pl.ds(ib0, IB_W)], ids_bufs[0], in_sems[0])
    for g in range(CB):
        b = g & 1
        ob = g % 3
        in_cp[g].wait()
        if g + 1 < CB:
            in_cp[g + 1] = pltpu.async_copy(
                ids_hbm.at[pl.ds(g + 1, 1), pl.ds(ib0, IB_W)],
                ids_bufs[1 - b], in_sems[1 - b])
        if g >= 3:
            out_cp[g - 3].wait()
        _compute_chunk(ids_bufs[b], out_bufs[ob], tab_v)
        out_cp[g] = pltpu.async_copy(
            out_bufs[ob],
            out_hbm.at[pl.ds(g * 8, 8), pl.ds(ib0, IB_W)],
            out_sems[ob])
    out_cp[CB - 3].wait()
    out_cp[CB - 2].wait()
    out_cp[CB - 1].wait()


_sc_lookup = functools.partial(
    pl.kernel,
    out_type=jax.ShapeDtypeStruct((C, IB_ALL, D, 128), jnp.float32),
    mesh=plsc.VectorSubcoreMesh(
        core_axis_name="c", subcore_axis_name="s",
        num_cores=NC, num_subcores=NS),
    scratch_types=[
        pltpu.VMEM((8 * D,), jnp.float32),
        pltpu.VMEM((1, IB_W, 8, 128), jnp.int32),
        pltpu.VMEM((1, IB_W, 8, 128), jnp.int32),
        pltpu.VMEM((8, IB_W, D, 128), jnp.float32),
        pltpu.VMEM((8, IB_W, D, 128), jnp.float32),
        pltpu.VMEM((8, IB_W, D, 128), jnp.float32),
        pltpu.SemaphoreType.DMA,
        pltpu.SemaphoreType.DMA,
        pltpu.SemaphoreType.DMA,
        pltpu.SemaphoreType.DMA,
        pltpu.SemaphoreType.DMA,
    ],
    compiler_params=pltpu.CompilerParams(
        use_tc_tiling_on_sc=False, needs_layout_passes=False),
)(_body)


def kernel(input_ids, embed_tokens):
    ids4 = (input_ids.astype(jnp.int32).T
            .reshape(CB, 8, IB_ALL, 128).transpose(0, 2, 1, 3))
    tab_flat = (embed_tokens.astype(jnp.float32) + jnp.float32(0.02)
                ).reshape(8 * D)
    z = _sc_lookup(ids4, tab_flat)
    return jnp.transpose(z, (1, 3, 0, 2)).reshape(R, C, D)
